# Initial kernel scaffold; baseline (speedup 1.0000x reference)
import jax
import jax.numpy as jnp
from jax.experimental import pallas as pl


def kernel(boxes, scores, idxs):
    raise NotImplementedError("write your pallas kernel here")


# speedup vs baseline: 49.4116x; 49.4116x over previous
import functools

import jax
import jax.numpy as jnp
from jax.experimental import pallas as pl
from jax.experimental.pallas import tpu as pltpu

_IOU_T = 0.7
_MAX_OUT = 256
_B = 128


def _overlap_mask(x1c, y1c, x2c, y2c, x1r, y1r, x2r, y2r, area_r):
    area_c = (x2c - x1c + 1.0) * (y2c - y1c + 1.0)
    xx1 = jnp.maximum(x1c, x1r)
    yy1 = jnp.maximum(y1c, y1r)
    xx2 = jnp.minimum(x2c, x2r)
    yy2 = jnp.minimum(y2c, y2r)
    w = jnp.maximum(0.0, xx2 - xx1 + 1.0)
    h = jnp.maximum(0.0, yy2 - yy1 + 1.0)
    inter = w * h
    denom = jnp.minimum(area_c, area_r)
    o = inter / denom
    return o > _IOU_T


def _nms_body(nblk, np_, data_ref, dataT_ref, out_ref, misc_ref, keep_ref):
    data = data_ref[...]
    x1r = data[0:1, :]
    y1r = data[1:2, :]
    x2r = data[2:3, :]
    y2r = data[3:4, :]
    area_r = (x2r - x1r + 1.0) * (y2r - y1r + 1.0)
    keep_ref[...] = data[10:11, :]

    def block_step(b, _):
        blk = dataT_ref[pl.ds(b * _B, _B), :]
        x1c = blk[:, 0:1]
        y1c = blk[:, 1:2]
        x2c = blk[:, 2:3]
        y2c = blk[:, 3:4]
        jcol = jax.lax.broadcasted_iota(jnp.int32, (_B, np_), 1)
        irow = jax.lax.broadcasted_iota(jnp.int32, (_B, np_), 0) + b * _B
        om = _overlap_mask(x1c, y1c, x2c, y2c, x1r, y1r, x2r, y2r, area_r)
        mf = jnp.where(om & (jcol > irow), 1.0, 0.0)

        x1rb = data_ref[0:1, pl.ds(b * _B, _B)]
        y1rb = data_ref[1:2, pl.ds(b * _B, _B)]
        x2rb = data_ref[2:3, pl.ds(b * _B, _B)]
        y2rb = data_ref[3:4, pl.ds(b * _B, _B)]
        area_rb = (x2rb - x1rb + 1.0) * (y2rb - y1rb + 1.0)
        omb = _overlap_mask(x1c, y1c, x2c, y2c, x1rb, y1rb, x2rb, y2rb,
                            area_rb)
        jcb = jax.lax.broadcasted_iota(jnp.int32, (_B, _B), 1)
        irb = jax.lax.broadcasted_iota(jnp.int32, (_B, _B), 0)
        mintra = jnp.where(omb & (jcb > irb), 1.0, 0.0)

        kinit = keep_ref[0:1, pl.ds(b * _B, _B)]

        def cond(c):
            kp, k = c
            return jnp.any(kp != k)

        def body(c):
            _, k = c
            supp = jax.lax.dot_general(
                k, mintra, (((1,), (0,)), ((), ())),
                precision=jax.lax.Precision.HIGHEST,
                preferred_element_type=jnp.float32)
            knew = jnp.where(supp > 0.5, 0.0, kinit)
            return (k, knew)

        _, kfin = jax.lax.while_loop(cond, body, (kinit - 1.0, kinit))

        supp_all = jax.lax.dot_general(
            kfin, mf, (((1,), (0,)), ((), ())),
            precision=jax.lax.Precision.HIGHEST,
            preferred_element_type=jnp.float32)
        keep_ref[...] = jnp.where(supp_all > 0.5, 0.0, keep_ref[...])
        return 0

    jax.lax.fori_loop(0, nblk, block_step, 0)

    keep = keep_ref[...]
    count = jnp.sum(keep)
    c = keep
    s = 1
    while s < np_:
        shifted = jnp.concatenate(
            [jnp.zeros((1, s), jnp.float32), c[:, : np_ - s]], axis=1)
        c = c + shifted
        s *= 2
    sval = (jax.lax.broadcasted_iota(jnp.int32, (_MAX_OUT, 1), 0) + 1
            ).astype(jnp.float32)
    onehot = jnp.where((c == sval) & (keep > 0.5), 1.0, 0.0)

    def gath(row):
        return jnp.sum(onehot * row, axis=1, keepdims=True)

    vc = jnp.where(
        jax.lax.broadcasted_iota(jnp.int32, (_MAX_OUT, 1), 0).astype(
            jnp.float32) < count,
        1.0, 0.0)
    pk = gath(data[9:10, :])
    pk = jnp.where(vc > 0.5, pk, data[9:10, 0:1])

    out_ref[:, 0:1] = gath(data[4:5, :])
    out_ref[:, 1:2] = gath(data[5:6, :])
    out_ref[:, 2:3] = gath(data[6:7, :])
    out_ref[:, 3:4] = gath(data[7:8, :])
    out_ref[:, 4:5] = gath(data[8:9, :])
    out_ref[:, 5:8] = jnp.zeros((_MAX_OUT, 3), jnp.float32)
    misc_ref[:, 0:1] = pk
    misc_ref[:, 1:2] = vc
    misc_ref[:, 2:8] = jnp.zeros((_MAX_OUT, 6), jnp.float32)


def kernel(boxes, scores, idxs):
    n = boxes.shape[0]
    nblk = (n + _B - 1) // _B
    np_ = nblk * _B

    max_coordinate = boxes.max()
    offsets = idxs.astype(boxes.dtype) * (max_coordinate + 1.0)
    boxes_for_nms = boxes + offsets[:, None]
    order = jnp.argsort(-scores)
    bo = boxes_for_nms[order]
    bb = boxes[order]
    ss = scores[order]
    orderf = order.astype(jnp.float32)

    pad = np_ - n
    rows = [bo[:, 0], bo[:, 1], bo[:, 2], bo[:, 3],
            bb[:, 0], bb[:, 1], bb[:, 2], bb[:, 3],
            ss, orderf, jnp.ones((n,), jnp.float32)]
    data = jnp.stack([jnp.pad(r, (0, pad)) for r in rows], axis=0)
    dataT = data.T

    out8, misc = pl.pallas_call(
        functools.partial(_nms_body, nblk, np_),
        out_shape=[
            jax.ShapeDtypeStruct((_MAX_OUT, 8), jnp.float32),
            jax.ShapeDtypeStruct((_MAX_OUT, 8), jnp.float32),
        ],
        scratch_shapes=[pltpu.VMEM((1, np_), jnp.float32)],
    )(data, dataT)

    out = out8[:, :5]
    picks = misc[:, 0].astype(jnp.int32)
    valid = misc[:, 1] > 0.5
    return out, picks, valid


# speedup vs baseline: 62.9920x; 1.2748x over previous
import functools

import jax
import jax.numpy as jnp
from jax.experimental import pallas as pl
from jax.experimental.pallas import tpu as pltpu

_IOU_T = 0.7
_MAX_OUT = 256
_B = 128
_W = 1024


def _overlap_mask(x1c, y1c, x2c, y2c, x1r, y1r, x2r, y2r):
    area_c = (x2c - x1c + 1.0) * (y2c - y1c + 1.0)
    area_r = (x2r - x1r + 1.0) * (y2r - y1r + 1.0)
    xx1 = jnp.maximum(x1c, x1r)
    yy1 = jnp.maximum(y1c, y1r)
    xx2 = jnp.minimum(x2c, x2r)
    yy2 = jnp.minimum(y2c, y2r)
    w = jnp.maximum(0.0, xx2 - xx1 + 1.0)
    h = jnp.maximum(0.0, yy2 - yy1 + 1.0)
    inter = w * h
    denom = jnp.minimum(area_c, area_r)
    o = inter / denom
    return o > _IOU_T


def _nms_body(nblk, np_, npad, data_ref, dataT_ref, out_ref, misc_ref,
              keep_ref):
    keep_ref[...] = data_ref[10:11, :]

    def block_step(b, _):
        base = b * _B
        blk = dataT_ref[pl.ds(base, _B), :]
        x1c = blk[:, 0:1]
        y1c = blk[:, 1:2]
        x2c = blk[:, 2:3]
        y2c = blk[:, 3:4]
        irow = jax.lax.broadcasted_iota(jnp.int32, (_B, _W), 0) + base

        def chunk_mask(start):
            x1r = data_ref[0:1, pl.ds(start, _W)]
            y1r = data_ref[1:2, pl.ds(start, _W)]
            x2r = data_ref[2:3, pl.ds(start, _W)]
            y2r = data_ref[3:4, pl.ds(start, _W)]
            om = _overlap_mask(x1c, y1c, x2c, y2c, x1r, y1r, x2r, y2r)
            jcol = jax.lax.broadcasted_iota(jnp.int32, (_B, _W), 1) + start
            return jnp.where(om & (jcol > irow), 1.0, 0.0)

        def apply_chunk(kfin, start, mf):
            supp = jax.lax.dot_general(
                kfin, mf, (((1,), (0,)), ((), ())),
                precision=jax.lax.Precision.HIGHEST,
                preferred_element_type=jnp.float32)
            cur = keep_ref[0:1, pl.ds(start, _W)]
            keep_ref[0:1, pl.ds(start, _W)] = jnp.where(
                supp > 0.5, 0.0, cur)

        mf0 = chunk_mask(base)
        mintra = mf0[:, 0:_B]
        kinit = keep_ref[0:1, pl.ds(base, _B)]

        def cond(c):
            kp, k = c
            return jnp.any(kp != k)

        def body(c):
            _, k = c
            supp = jax.lax.dot_general(
                k, mintra, (((1,), (0,)), ((), ())),
                precision=jax.lax.Precision.HIGHEST,
                preferred_element_type=jnp.float32)
            knew = jnp.where(supp > 0.5, 0.0, kinit)
            return (k, knew)

        _, kfin = jax.lax.while_loop(cond, body, (kinit - 1.0, kinit))

        apply_chunk(kfin, base, mf0)
        nc = (np_ - base + _W - 1) // _W

        def chunk_step(c, _):
            start = base + c * _W
            apply_chunk(kfin, start, chunk_mask(start))
            return 0

        jax.lax.fori_loop(1, nc, chunk_step, 0)
        return 0

    jax.lax.fori_loop(0, nblk, block_step, 0)

    keep = keep_ref[0:1, 0:np_]
    data = data_ref[...]
    count = jnp.sum(keep)
    c = keep
    s = 1
    while s < np_:
        shifted = jnp.concatenate(
            [jnp.zeros((1, s), jnp.float32), c[:, : np_ - s]], axis=1)
        c = c + shifted
        s *= 2
    sval = (jax.lax.broadcasted_iota(jnp.int32, (_MAX_OUT, 1), 0) + 1
            ).astype(jnp.float32)
    onehot = jnp.where((c == sval) & (keep > 0.5), 1.0, 0.0)

    def gath(row):
        return jnp.sum(onehot * row[:, 0:np_], axis=1, keepdims=True)

    vc = jnp.where(
        jax.lax.broadcasted_iota(jnp.int32, (_MAX_OUT, 1), 0).astype(
            jnp.float32) < count,
        1.0, 0.0)
    pk = gath(data[9:10, :])
    pk = jnp.where(vc > 0.5, pk, data[9:10, 0:1])

    out_ref[:, 0:1] = gath(data[4:5, :])
    out_ref[:, 1:2] = gath(data[5:6, :])
    out_ref[:, 2:3] = gath(data[6:7, :])
    out_ref[:, 3:4] = gath(data[7:8, :])
    out_ref[:, 4:5] = gath(data[8:9, :])
    out_ref[:, 5:8] = jnp.zeros((_MAX_OUT, 3), jnp.float32)
    misc_ref[:, 0:1] = pk
    misc_ref[:, 1:2] = vc
    misc_ref[:, 2:8] = jnp.zeros((_MAX_OUT, 6), jnp.float32)


def kernel(boxes, scores, idxs):
    n = boxes.shape[0]
    nblk = (n + _B - 1) // _B
    np_ = nblk * _B
    npad = np_ + _W

    max_coordinate = boxes.max()
    offsets = idxs.astype(boxes.dtype) * (max_coordinate + 1.0)
    boxes_for_nms = boxes + offsets[:, None]
    order = jnp.argsort(-scores)
    bo = boxes_for_nms[order]
    bb = boxes[order]
    ss = scores[order]
    orderf = order.astype(jnp.float32)

    pad = npad - n
    rows = [bo[:, 0], bo[:, 1], bo[:, 2], bo[:, 3],
            bb[:, 0], bb[:, 1], bb[:, 2], bb[:, 3],
            ss, orderf, jnp.ones((n,), jnp.float32)]
    data = jnp.stack([jnp.pad(r, (0, pad)) for r in rows], axis=0)
    dataT = data.T

    out8, misc = pl.pallas_call(
        functools.partial(_nms_body, nblk, np_, npad),
        out_shape=[
            jax.ShapeDtypeStruct((_MAX_OUT, 8), jnp.float32),
            jax.ShapeDtypeStruct((_MAX_OUT, 8), jnp.float32),
        ],
        scratch_shapes=[pltpu.VMEM((1, npad), jnp.float32)],
    )(data, dataT)

    out = out8[:, :5]
    picks = misc[:, 0].astype(jnp.int32)
    valid = misc[:, 1] > 0.5
    return out, picks, valid
